# baseline (speedup 1.0000x reference)
import jax
import jax.numpy as jnp
from jax.experimental import pallas as pl
from jax.experimental.pallas import tpu as pltpu

_D_MODEL = 768
_D_FF = 3072
_K = 512
_CODE_DIM = 32
_BETA = 1.0

_BR1 = 32
_BR2 = 128
_NQ = 768 // _BR1
_RB = 1024
_NM = 4096 // _RB


def _soft_assign_128(v, c4m, cout, onesm, c2, nrows):
    g16 = v.reshape(nrows, 128).astype(jnp.bfloat16)
    acc = jnp.zeros((nrows, 128), jnp.float32)
    srec = jnp.zeros((nrows, 128), jnp.float32)
    for o in range(4):
        logits = jnp.dot(
            g16, c4m[:, _K * o : _K * (o + 1)], preferred_element_type=jnp.float32
        )
        e16 = jnp.exp((logits - c2).astype(jnp.bfloat16))
        s = jnp.sum(e16, axis=1, keepdims=True, dtype=jnp.float32)
        acc = acc + jnp.dot(
            e16, cout[_K * o : _K * (o + 1), :], preferred_element_type=jnp.float32
        )
        srec = srec + jnp.dot(
            s, onesm[o : o + 1, :], preferred_element_type=jnp.float32
        )
    return acc / srec


def _fused_body(g1_ref, g2_ref, b1_ref, b2_ref, cen_ref, x_ref, y_ref,
                qw1_ref, qw2_ref, qb_ref, c4m_ref, cout_ref, onesm_ref, c2_ref):
    i = pl.program_id(0)

    @pl.when(i == 0)
    def _setup_step():
        cen = cen_ref[...]
        ct = jnp.transpose(cen)
        ct16 = ((2.0 * _BETA) * ct).astype(jnp.bfloat16)
        c4m_ref[...] = jnp.concatenate(
            [jnp.pad(ct16, ((32 * o, 96 - 32 * o), (0, 0))) for o in range(4)],
            axis=1,
        )
        cen16 = cen.astype(jnp.bfloat16)
        cout_ref[...] = jnp.concatenate(
            [jnp.pad(cen16, ((0, 0), (32 * o, 96 - 32 * o))) for o in range(4)],
            axis=0,
        )
        c2_ref[...] = _BETA * jnp.sum(ct * ct, axis=0, keepdims=True)
        lane = jax.lax.broadcasted_iota(jnp.int32, (4, 128), 1)
        off = jax.lax.broadcasted_iota(jnp.int32, (4, 128), 0)
        onesm_ref[...] = ((lane >= 32 * off) & (lane < 32 * (off + 1))).astype(
            jnp.float32
        )
        gbv = jnp.concatenate(
            [
                b1_ref[...].reshape(24, 128),
                b2_ref[...].reshape(6, 128),
                jnp.zeros((2, 128), jnp.float32),
            ],
            axis=0,
        )
        qb_ref[...] = _soft_assign_128(
            gbv, c4m_ref[...], cout_ref[...], onesm_ref[...], c2_ref[...], 32
        )

    @pl.when(i < _NQ)
    def _quantize_step():
        c4m = c4m_ref[...]
        cout = cout_ref[...]
        onesm = onesm_ref[...]
        c2 = c2_ref[...]
        n1 = _BR1 * _D_FF // 128
        n2 = _BR2 * _D_MODEL // 128
        gcat = jnp.concatenate(
            [g1_ref[...].reshape(n1, 128), g2_ref[...].reshape(n2, 128)], axis=0
        )
        q = _soft_assign_128(gcat, c4m, cout, onesm, c2, n1 + n2)
        qw1_ref[pl.ds(i * _BR1, _BR1), :] = q[:n1, :].reshape(_BR1, _D_FF).astype(
            jnp.bfloat16
        )
        qw2_ref[pl.ds(i * _BR2, _BR2), :] = (
            q[n1:, :].reshape(_BR2, _D_MODEL).astype(jnp.bfloat16)
        )

    @pl.when(i >= _NQ)
    def _mlp_step():
        qb = qb_ref[...]
        b1 = qb[0:24, :].reshape(1, _D_FF)
        b2 = qb[24:30, :].reshape(1, _D_MODEL)
        h = jnp.dot(
            x_ref[...].astype(jnp.bfloat16),
            qw1_ref[...],
            preferred_element_type=jnp.float32,
        )
        h = jnp.maximum(h + b1, 0.0)
        y = jnp.dot(
            h.astype(jnp.bfloat16), qw2_ref[...], preferred_element_type=jnp.float32
        )
        y_ref[...] = y + b2


def kernel(x, W1, b1, W2, b2, centroids):
    xf = x.reshape(-1, _D_MODEL)
    y = pl.pallas_call(
        _fused_body,
        grid=(_NQ + _NM,),
        in_specs=[
            pl.BlockSpec((_BR1, _D_FF), lambda i: (jnp.minimum(i, _NQ - 1), 0)),
            pl.BlockSpec((_BR2, _D_MODEL), lambda i: (jnp.minimum(i, _NQ - 1), 0)),
            pl.BlockSpec((1, _D_FF), lambda i: (0, 0)),
            pl.BlockSpec((1, _D_MODEL), lambda i: (0, 0)),
            pl.BlockSpec((_K, _CODE_DIM), lambda i: (0, 0)),
            pl.BlockSpec((_RB, _D_MODEL), lambda i: (jnp.maximum(i - _NQ, 0), 0)),
        ],
        out_specs=pl.BlockSpec((_RB, _D_MODEL), lambda i: (jnp.maximum(i - _NQ, 0), 0)),
        out_shape=jax.ShapeDtypeStruct((4096, _D_MODEL), jnp.float32),
        scratch_shapes=[
            pltpu.VMEM((_D_MODEL, _D_FF), jnp.bfloat16),
            pltpu.VMEM((_D_FF, _D_MODEL), jnp.bfloat16),
            pltpu.VMEM((32, 128), jnp.float32),
            pltpu.VMEM((128, 4 * _K), jnp.bfloat16),
            pltpu.VMEM((4 * _K, 128), jnp.bfloat16),
            pltpu.VMEM((4, 128), jnp.float32),
            pltpu.VMEM((1, _K), jnp.float32),
        ],
    )(W1, W2, b1.reshape(1, _D_FF), b2.reshape(1, _D_MODEL), centroids, xf)
    return y.reshape(x.shape)


# speedup vs baseline: 1.0274x; 1.0274x over previous
import jax
import jax.numpy as jnp
from jax.experimental import pallas as pl
from jax.experimental.pallas import tpu as pltpu

_D_MODEL = 768
_D_FF = 3072
_K = 512
_CODE_DIM = 32
_BETA = 1.0

_BR1 = 128
_BR2 = 512
_NQ = 768 // _BR1
_RB = 1024
_NM = 4096 // _RB


def _soft_assign_128(v, c4m, cout, onesm, c2, nrows):
    g16 = v.reshape(nrows, 128).astype(jnp.bfloat16)
    acc = jnp.zeros((nrows, 128), jnp.float32)
    srec = jnp.zeros((nrows, 128), jnp.float32)
    for o in range(4):
        logits = jnp.dot(
            g16, c4m[:, _K * o : _K * (o + 1)], preferred_element_type=jnp.float32
        )
        e16 = jnp.exp((logits - c2).astype(jnp.bfloat16))
        s = jnp.sum(e16, axis=1, keepdims=True, dtype=jnp.float32)
        acc = acc + jnp.dot(
            e16, cout[_K * o : _K * (o + 1), :], preferred_element_type=jnp.float32
        )
        srec = srec + jnp.dot(
            s, onesm[o : o + 1, :], preferred_element_type=jnp.float32
        )
    return acc / srec


def _fused_body(g1_ref, g2_ref, b1_ref, b2_ref, cen_ref, x_ref, y_ref,
                qw1_ref, qw2_ref, qb_ref, c4m_ref, cout_ref, onesm_ref, c2_ref):
    i = pl.program_id(0)

    @pl.when(i == 0)
    def _setup_step():
        cen = cen_ref[...]
        ct = jnp.transpose(cen)
        ct16 = ((2.0 * _BETA) * ct).astype(jnp.bfloat16)
        c4m_ref[...] = jnp.concatenate(
            [jnp.pad(ct16, ((32 * o, 96 - 32 * o), (0, 0))) for o in range(4)],
            axis=1,
        )
        cen16 = cen.astype(jnp.bfloat16)
        cout_ref[...] = jnp.concatenate(
            [jnp.pad(cen16, ((0, 0), (32 * o, 96 - 32 * o))) for o in range(4)],
            axis=0,
        )
        c2_ref[...] = _BETA * jnp.sum(ct * ct, axis=0, keepdims=True)
        lane = jax.lax.broadcasted_iota(jnp.int32, (4, 128), 1)
        off = jax.lax.broadcasted_iota(jnp.int32, (4, 128), 0)
        onesm_ref[...] = ((lane >= 32 * off) & (lane < 32 * (off + 1))).astype(
            jnp.float32
        )
        gbv = jnp.concatenate(
            [
                b1_ref[...].reshape(24, 128),
                b2_ref[...].reshape(6, 128),
                jnp.zeros((2, 128), jnp.float32),
            ],
            axis=0,
        )
        qb_ref[...] = _soft_assign_128(
            gbv, c4m_ref[...], cout_ref[...], onesm_ref[...], c2_ref[...], 32
        )

    @pl.when(i < _NQ)
    def _quantize_step():
        c4m = c4m_ref[...]
        cout = cout_ref[...]
        onesm = onesm_ref[...]
        c2 = c2_ref[...]
        n1 = _BR1 * _D_FF // 128
        n2 = _BR2 * _D_MODEL // 128
        gcat = jnp.concatenate(
            [g1_ref[...].reshape(n1, 128), g2_ref[...].reshape(n2, 128)], axis=0
        )
        q = _soft_assign_128(gcat, c4m, cout, onesm, c2, n1 + n2)
        qw1_ref[pl.ds(i * _BR1, _BR1), :] = q[:n1, :].reshape(_BR1, _D_FF).astype(
            jnp.bfloat16
        )
        qw2_ref[pl.ds(i * _BR2, _BR2), :] = (
            q[n1:, :].reshape(_BR2, _D_MODEL).astype(jnp.bfloat16)
        )

    @pl.when(i >= _NQ)
    def _mlp_step():
        qb = qb_ref[...]
        b1 = qb[0:24, :].reshape(1, _D_FF)
        b2 = qb[24:30, :].reshape(1, _D_MODEL)
        h = jnp.dot(
            x_ref[...].astype(jnp.bfloat16),
            qw1_ref[...],
            preferred_element_type=jnp.float32,
        )
        h = jnp.maximum(h + b1, 0.0)
        y = jnp.dot(
            h.astype(jnp.bfloat16), qw2_ref[...], preferred_element_type=jnp.float32
        )
        y_ref[...] = y + b2


def kernel(x, W1, b1, W2, b2, centroids):
    xf = x.reshape(-1, _D_MODEL)
    y = pl.pallas_call(
        _fused_body,
        grid=(_NQ + _NM,),
        in_specs=[
            pl.BlockSpec((_BR1, _D_FF), lambda i: (jnp.minimum(i, _NQ - 1), 0)),
            pl.BlockSpec((_BR2, _D_MODEL), lambda i: (jnp.minimum(i, _NQ - 1), 0)),
            pl.BlockSpec((1, _D_FF), lambda i: (0, 0)),
            pl.BlockSpec((1, _D_MODEL), lambda i: (0, 0)),
            pl.BlockSpec((_K, _CODE_DIM), lambda i: (0, 0)),
            pl.BlockSpec((_RB, _D_MODEL), lambda i: (jnp.maximum(i - _NQ, 0), 0)),
        ],
        out_specs=pl.BlockSpec((_RB, _D_MODEL), lambda i: (jnp.maximum(i - _NQ, 0), 0)),
        out_shape=jax.ShapeDtypeStruct((4096, _D_MODEL), jnp.float32),
        scratch_shapes=[
            pltpu.VMEM((_D_MODEL, _D_FF), jnp.bfloat16),
            pltpu.VMEM((_D_FF, _D_MODEL), jnp.bfloat16),
            pltpu.VMEM((32, 128), jnp.float32),
            pltpu.VMEM((128, 4 * _K), jnp.bfloat16),
            pltpu.VMEM((4 * _K, 128), jnp.bfloat16),
            pltpu.VMEM((4, 128), jnp.float32),
            pltpu.VMEM((1, _K), jnp.float32),
        ],
    )(W1, W2, b1.reshape(1, _D_FF), b2.reshape(1, _D_MODEL), centroids, xf)
    return y.reshape(x.shape)
